# baseline (speedup 1.0000x reference)
import functools

import jax
import jax.numpy as jnp
from jax import lax
from jax.experimental import pallas as pl
from jax.experimental.pallas import tpu as pltpu
from jax.experimental.pallas import tpu_sc as plsc

_B = 16384
_P = 128
_NP = 127
_D = _P * 3
_NC = 2
_NS = 16
_NW = _NC * _NS
_RPW = _B // _NW
_CHUNK = 32
_NCHUNK = _RPW // _CHUNK
_L = 16
_NG = 8


def _group_start(g):
    return _NP - _L if g == _NG - 1 else _L * g


def _rsqrt_nr(s):
    ib = lax.bitcast_convert_type(s, jnp.int32)
    ib = jnp.int32(0x5F3759DF) - lax.shift_right_logical(ib, 1)
    r = lax.bitcast_convert_type(ib, jnp.float32)
    for _ in range(3):
        r = r * (1.5 - 0.5 * s * r * r)
    return r


def _sc_body(x_hbm, idx_hbm, out_hbm, idxbuf, colbuf, xbuf, obuf):
    wid = lax.axis_index("s") * _NC + lax.axis_index("c")
    iota = lax.iota(jnp.int32, _L)
    zeros = jnp.full((_L,), 0, jnp.int32)

    pltpu.sync_copy(idx_hbm, idxbuf)
    for g in range(_NG):
        rows = _group_start(g) + iota
        i1 = plsc.load_gather(idxbuf, [rows * 2])
        i2 = plsc.load_gather(idxbuf, [rows * 2 + 1])
        colbuf[pl.ds((2 * g) * _L, _L)] = i1 * 3
        colbuf[pl.ds((2 * g + 1) * _L, _L)] = i2 * 3

    def do_chunk(k, carry):
        base = wid * _RPW + k * _CHUNK
        pltpu.sync_copy(x_hbm.at[pl.ds(base * _D, _CHUNK * _D)], xbuf)

        def do_row(r, carry2):
            rbase = zeros + r * _D
            for g in range(_NG):
                c1 = rbase + colbuf[pl.ds((2 * g) * _L, _L)]
                c2 = rbase + colbuf[pl.ds((2 * g + 1) * _L, _L)]
                dx = (plsc.load_gather(xbuf, [c2])
                      - plsc.load_gather(xbuf, [c1]))
                dy = (plsc.load_gather(xbuf, [c2 + 1])
                      - plsc.load_gather(xbuf, [c1 + 1]))
                dz = (plsc.load_gather(xbuf, [c2 + 2])
                      - plsc.load_gather(xbuf, [c1 + 2]))
                s = dx * dx + dy * dy + dz * dz
                dist = jnp.where(s > 0.0, s * _rsqrt_nr(s), 0.0)
                obuf[pl.ds(r * _NP + _group_start(g), _L)] = dist
            return carry2

        lax.fori_loop(0, _CHUNK, do_row, 0)
        pltpu.sync_copy(obuf, out_hbm.at[pl.ds(base * _NP, _CHUNK * _NP)])
        return carry

    lax.fori_loop(0, _NCHUNK, do_chunk, 0)


_sc_distances = functools.partial(
    pl.kernel,
    out_type=jax.ShapeDtypeStruct((_B * _NP,), jnp.float32),
    mesh=plsc.VectorSubcoreMesh(
        core_axis_name="c", subcore_axis_name="s",
        num_cores=_NC, num_subcores=_NS),
    compiler_params=pltpu.CompilerParams(needs_layout_passes=False),
    scratch_types=[
        pltpu.VMEM((2 * _P,), jnp.int32),
        pltpu.VMEM((2 * _NG * _L,), jnp.int32),
        pltpu.VMEM((_CHUNK * _D,), jnp.float32),
        pltpu.VMEM((_CHUNK * _NP,), jnp.float32),
    ],
)(_sc_body)


def kernel(x, idx):
    x1d = x.reshape(_B * _D)
    idxp = jnp.pad(idx, ((0, _P - _NP), (0, 0))).reshape(2 * _P)
    return _sc_distances(x1d, idxp).reshape(_B, _NP)
